# baseline (speedup 1.0000x reference)
import jax
import jax.numpy as jnp
from jax.experimental import pallas as pl
from jax.experimental.pallas import tpu as pltpu

N = 100000
D = 64
EPS = 1e-5
BLK = 2000
C = N // BLK


def _stats_body(x_ref, o_ref, acc_ref):
    i = pl.program_id(0)

    @pl.when(i == 0)
    def _():
        acc_ref[...] = jnp.zeros_like(acc_ref)

    x = x_ref[...]
    acc_ref[0:1, :] += jnp.sum(x, axis=0, keepdims=True)
    acc_ref[1:2, :] += jnp.sum(x * x, axis=0, keepdims=True)

    @pl.when(i == C - 1)
    def _():
        mean = acc_ref[0:1, :] / N
        ex2 = acc_ref[1:2, :] / N
        var = ex2 - mean * mean
        rstd = jax.lax.rsqrt(var + EPS)
        o_ref[...] = jnp.concatenate([mean, rstd], axis=0)


def _norm_body(x_ref, st_ref, o_ref):
    mean = st_ref[0:1, :]
    rstd = st_ref[1:2, :]
    o_ref[...] = ((x_ref[...] - mean) * rstd)[None, :, None, :]


def kernel(table):
    stats = pl.pallas_call(
        _stats_body,
        grid=(C,),
        in_specs=[pl.BlockSpec((BLK, D), lambda i: (i, 0))],
        out_specs=pl.BlockSpec((2, D), lambda i: (0, 0)),
        out_shape=jax.ShapeDtypeStruct((2, D), jnp.float32),
        scratch_shapes=[pltpu.VMEM((2, D), jnp.float32)],
    )(table)
    normed = pl.pallas_call(
        _norm_body,
        grid=(C,),
        in_specs=[
            pl.BlockSpec((BLK, D), lambda i: (i, 0)),
            pl.BlockSpec((2, D), lambda i: (0, 0)),
        ],
        out_specs=pl.BlockSpec((1, BLK, 1, D), lambda i: (0, i, 0, 0)),
        out_shape=jax.ShapeDtypeStruct((1, N, 1, D), jnp.float32),
    )(table, stats)
    return normed


# speedup vs baseline: 4.8993x; 4.8993x over previous
import jax
import jax.numpy as jnp
from jax.experimental import pallas as pl
from jax.experimental.pallas import tpu as pltpu

N = 100000
D = 64
EPS = 1e-5
BLK = 4096
C = pl.cdiv(N, BLK)


def _stats_body(x_ref, mean_ref, rstd_ref, acc_ref, accq_ref):
    i = pl.program_id(0)

    @pl.when(i == 0)
    def _():
        acc_ref[...] = jnp.zeros_like(acc_ref)
        accq_ref[...] = jnp.zeros_like(accq_ref)

    x = x_ref[...]
    lane = jax.lax.broadcasted_iota(jnp.int32, (D, BLK), 1)
    valid = (i * BLK + lane) < N
    x = jnp.where(valid, x, 0.0)
    acc_ref[...] += x
    accq_ref[...] += x * x

    @pl.when(i == C - 1)
    def _():
        s = jnp.sum(acc_ref[...], axis=1, keepdims=True)
        q = jnp.sum(accq_ref[...], axis=1, keepdims=True)
        mean = s / N
        var = q / N - mean * mean
        rstd = jax.lax.rsqrt(var + EPS)
        mean_ref[...] = jnp.broadcast_to(mean, (D, 128))
        rstd_ref[...] = jnp.broadcast_to(rstd, (D, 128))


def _norm_body(x_ref, mean_ref, rstd_ref, o_ref):
    mean = mean_ref[:, 0:1]
    rstd = rstd_ref[:, 0:1]
    o_ref[...] = (x_ref[...] - mean) * rstd


def kernel(table):
    tt = table.T
    mean, rstd = pl.pallas_call(
        _stats_body,
        grid=(C,),
        in_specs=[pl.BlockSpec((D, BLK), lambda i: (0, i))],
        out_specs=[
            pl.BlockSpec((D, 128), lambda i: (0, 0)),
            pl.BlockSpec((D, 128), lambda i: (0, 0)),
        ],
        out_shape=[
            jax.ShapeDtypeStruct((D, 128), jnp.float32),
            jax.ShapeDtypeStruct((D, 128), jnp.float32),
        ],
        scratch_shapes=[
            pltpu.VMEM((D, BLK), jnp.float32),
            pltpu.VMEM((D, BLK), jnp.float32),
        ],
    )(tt)
    normed = pl.pallas_call(
        _norm_body,
        grid=(C,),
        in_specs=[
            pl.BlockSpec((D, BLK), lambda i: (0, i)),
            pl.BlockSpec((D, 128), lambda i: (0, 0)),
            pl.BlockSpec((D, 128), lambda i: (0, 0)),
        ],
        out_specs=pl.BlockSpec((D, BLK), lambda i: (0, i)),
        out_shape=jax.ShapeDtypeStruct((D, N), jnp.float32),
    )(tt, mean, rstd)
    return normed.T[None, :, None, :]
